# Initial kernel scaffold; baseline (speedup 1.0000x reference)
import jax
import jax.numpy as jnp
from jax.experimental import pallas as pl


def kernel(x, edge_index, edge_attr, W_rel1, b_rel1, W_root1, W_rel2, b_rel2, W_root2):
    raise NotImplementedError("write your pallas kernel here")


# speedup vs baseline: 48.9257x; 48.9257x over previous
import functools

import jax
import jax.numpy as jnp
from jax import lax
from jax.experimental import pallas as pl
from jax.experimental.pallas import tpu as pltpu
from jax.experimental.pallas import tpu_sc as plsc

N = 100000
E = 3200000
F = 4
HID = 64
NC, NS = 2, 16
NW = NC * NS
E_PER_W = E // NW
CHUNK = 4000
K_CH = E_PER_W // CHUNK
SLICE_R = 6256

_MESH = plsc.VectorSubcoreMesh(
    core_axis_name="c", subcore_axis_name="s", num_cores=NC, num_subcores=NS)
_PARAMS = pltpu.CompilerParams(
    needs_layout_passes=False, use_tc_tiling_on_sc=False)


@functools.partial(
    pl.kernel,
    out_type=jax.ShapeDtypeStruct((NC, F, N), jnp.float32),
    mesh=_MESH,
    scratch_types=[
        [pltpu.VMEM_SHARED((N,), jnp.float32) for _ in range(F)],
        [pltpu.VMEM_SHARED((N,), jnp.float32) for _ in range(F)],
        pltpu.VMEM((CHUNK,), jnp.int32),
        pltpu.VMEM((CHUNK,), jnp.int32),
        pltpu.VMEM((CHUNK,), jnp.float32),
        [pltpu.VMEM((CHUNK,), jnp.float32) for _ in range(F)],
        pltpu.SemaphoreType.DMA,
    ],
    compiler_params=_PARAMS,
)
def _agg4(src_hbm, dst_hbm, w_hbm, x0, x1, x2, x3, zero_hbm, out_hbm,
          xtab, acc, src_v, dst_v, w_v, col_v, sem):
    c = lax.axis_index("c")
    s = lax.axis_index("s")
    wid = c * NS + s
    r0 = jnp.minimum(s * SLICE_R, N - SLICE_R)
    for f, xf in enumerate((x0, x1, x2, x3)):
        pltpu.sync_copy(xf.at[pl.ds(r0, SLICE_R)],
                        xtab[f].at[pl.ds(r0, SLICE_R)])
        pltpu.sync_copy(zero_hbm, acc[f].at[pl.ds(r0, SLICE_R)])
    plsc.subcore_barrier()

    base = wid * E_PER_W

    def chunk(k, carry):
        off = base + k * CHUNK
        pltpu.sync_copy(src_hbm.at[pl.ds(off, CHUNK)], src_v)
        pltpu.sync_copy(dst_hbm.at[pl.ds(off, CHUNK)], dst_v)
        pltpu.sync_copy(w_hbm.at[pl.ds(off, CHUNK)], w_v)
        for f in range(F):
            pltpu.async_copy(xtab[f].at[src_v], col_v[f], sem).wait()

        def scale(i, carry2):
            sl = pl.ds(i * 16, 16)
            wv = w_v[sl]
            for f in range(F):
                col_v[f][sl] = col_v[f][sl] * wv
            return carry2
        lax.fori_loop(0, CHUNK // 16, scale, 0)

        for f in range(F):
            pltpu.sync_copy(col_v[f], acc[f].at[dst_v], add=True)
        return carry
    lax.fori_loop(0, K_CH, chunk, 0)

    plsc.subcore_barrier()
    for f in range(F):
        pltpu.sync_copy(acc[f].at[pl.ds(r0, SLICE_R)],
                        out_hbm.at[c, f, pl.ds(r0, SLICE_R)])


@functools.partial(
    pl.kernel,
    out_type=jax.ShapeDtypeStruct((NC, N), jnp.float32),
    mesh=_MESH,
    scratch_types=[
        pltpu.VMEM_SHARED((N,), jnp.float32),
        pltpu.VMEM_SHARED((N,), jnp.float32),
        pltpu.VMEM((CHUNK,), jnp.int32),
        pltpu.VMEM((CHUNK,), jnp.int32),
        pltpu.VMEM((CHUNK,), jnp.float32),
        pltpu.VMEM((CHUNK,), jnp.float32),
        pltpu.SemaphoreType.DMA,
    ],
    compiler_params=_PARAMS,
)
def _agg1(src_hbm, dst_hbm, w_hbm, g_hbm, zero_hbm, out_hbm,
          gtab, acc, src_v, dst_v, w_v, m_v, sem):
    c = lax.axis_index("c")
    s = lax.axis_index("s")
    wid = c * NS + s
    r0 = jnp.minimum(s * SLICE_R, N - SLICE_R)
    pltpu.sync_copy(g_hbm.at[pl.ds(r0, SLICE_R)], gtab.at[pl.ds(r0, SLICE_R)])
    pltpu.sync_copy(zero_hbm, acc.at[pl.ds(r0, SLICE_R)])
    plsc.subcore_barrier()

    base = wid * E_PER_W

    def chunk(k, carry):
        off = base + k * CHUNK
        pltpu.sync_copy(src_hbm.at[pl.ds(off, CHUNK)], src_v)
        pltpu.sync_copy(dst_hbm.at[pl.ds(off, CHUNK)], dst_v)
        pltpu.sync_copy(w_hbm.at[pl.ds(off, CHUNK)], w_v)
        pltpu.async_copy(gtab.at[src_v], m_v, sem).wait()

        def scale(i, carry2):
            sl = pl.ds(i * 16, 16)
            m_v[sl] = m_v[sl] * w_v[sl]
            return carry2
        lax.fori_loop(0, CHUNK // 16, scale, 0)

        pltpu.sync_copy(m_v, acc.at[dst_v], add=True)
        return carry
    lax.fori_loop(0, K_CH, chunk, 0)

    plsc.subcore_barrier()
    pltpu.sync_copy(acc.at[pl.ds(r0, SLICE_R)],
                    out_hbm.at[c, pl.ds(r0, SLICE_R)])


_BLK = 5000


def _mid_body(p0, p1, x, w1, b1, wr1, w2, wr2, b2, g_out, rb_out):
    agg = p0[...] + p1[...]
    xv = x[...]
    w1v, wr1v = w1[...], wr1[...]
    pre = b1[...][None, :]
    for f in range(F):
        pre = pre + agg[:, f][:, None] * w1v[f, :][None, :]
        pre = pre + xv[:, f][:, None] * wr1v[f, :][None, :]
    h = jnp.maximum(pre, 0.0)
    g_out[...] = jnp.dot(h, w2[...], preferred_element_type=jnp.float32)
    rb_out[...] = (jnp.dot(h, wr2[...], preferred_element_type=jnp.float32)
                   + b2[...])


def _fin_body(s0, s1, rb, out):
    out[...] = s0[...] + s1[...] + rb[...]


def _row_spec(width):
    return pl.BlockSpec((_BLK, width), lambda i: (i, 0))


def _full_spec(shape):
    nd = len(shape)
    return pl.BlockSpec(shape, lambda i: (0,) * nd)


_mid = pl.pallas_call(
    _mid_body,
    grid=(N // _BLK,),
    in_specs=[
        _row_spec(F), _row_spec(F), _row_spec(F),
        _full_spec((F, HID)), _full_spec((HID,)), _full_spec((F, HID)),
        _full_spec((HID, 1)), _full_spec((HID, 1)), _full_spec((1,)),
    ],
    out_specs=[_row_spec(1), _row_spec(1)],
    out_shape=[jax.ShapeDtypeStruct((N, 1), jnp.float32),
               jax.ShapeDtypeStruct((N, 1), jnp.float32)],
)

_fin = pl.pallas_call(
    _fin_body,
    grid=(N // _BLK,),
    in_specs=[_row_spec(1), _row_spec(1), _row_spec(1)],
    out_specs=_row_spec(1),
    out_shape=jax.ShapeDtypeStruct((N, 1), jnp.float32),
)


def kernel(x, edge_index, edge_attr, W_rel1, b_rel1, W_root1,
           W_rel2, b_rel2, W_root2):
    src = edge_index[0].astype(jnp.int32)
    dst = edge_index[1].astype(jnp.int32)
    w = edge_attr.astype(jnp.float32)
    zero = jnp.zeros((SLICE_R,), jnp.float32)
    xcols = [x[:, f] for f in range(F)]

    p = _agg4(src, dst, w, *xcols, zero)
    g, rb = _mid(p[0].T, p[1].T, x, W_rel1, b_rel1, W_root1,
                 W_rel2, W_root2, b_rel2)
    sp = _agg1(src, dst, w, g.reshape(N), zero)
    return _fin(sp[0].reshape(N, 1), sp[1].reshape(N, 1), rb)


# speedup vs baseline: 60.5600x; 1.2378x over previous
import functools

import jax
import jax.numpy as jnp
from jax import lax
from jax.experimental import pallas as pl
from jax.experimental.pallas import tpu as pltpu
from jax.experimental.pallas import tpu_sc as plsc

N = 100000
E = 3200000
F = 4
HID = 64
NC, NS = 2, 16
NW = NC * NS
E_PER_W = E // NW
CHUNK = 2000
K_CH = E_PER_W // CHUNK
SLICE_R = 6256
BLK = 8192
GRID = 13
NP = BLK * GRID

_MESH = plsc.VectorSubcoreMesh(
    core_axis_name="c", subcore_axis_name="s", num_cores=NC, num_subcores=NS)
_PARAMS = pltpu.CompilerParams(
    needs_layout_passes=False, use_tc_tiling_on_sc=False)


def _zero_fill(zbuf):
    z16 = jnp.zeros((16,), jnp.float32)

    def zb(i, carry):
        zbuf[pl.ds(i * 16, 16)] = z16
        return carry
    lax.fori_loop(0, SLICE_R // 16, zb, 0)


@functools.partial(
    pl.kernel,
    out_type=jax.ShapeDtypeStruct((NC, F, N), jnp.float32),
    mesh=_MESH,
    scratch_types=[
        [pltpu.VMEM_SHARED((N,), jnp.float32) for _ in range(F)],
        [pltpu.VMEM_SHARED((N,), jnp.float32) for _ in range(F)],
        [pltpu.VMEM((CHUNK,), jnp.int32) for _ in range(2)],
        [pltpu.VMEM((CHUNK,), jnp.int32) for _ in range(2)],
        [pltpu.VMEM((CHUNK,), jnp.float32) for _ in range(2)],
        [[pltpu.VMEM((CHUNK,), jnp.float32) for _ in range(F)]
         for _ in range(2)],
        pltpu.VMEM((SLICE_R,), jnp.float32),
        pltpu.SemaphoreType.DMA,
        pltpu.SemaphoreType.DMA,
        pltpu.SemaphoreType.DMA,
    ],
    compiler_params=_PARAMS,
)
def _agg4(ei_hbm, w_hbm, x0, x1, x2, x3, out_hbm,
          xtab, acc, src_v, dst_v, w_v, col_v, zbuf, sem_ld, sem_g, sem_s):
    c = lax.axis_index("c")
    s = lax.axis_index("s")
    wid = c * NS + s
    r0 = jnp.minimum(s * SLICE_R, N - SLICE_R)
    _zero_fill(zbuf)
    stage = []
    for f, xf in enumerate((x0, x1, x2, x3)):
        stage.append(pltpu.async_copy(
            xf.at[pl.ds(r0, SLICE_R)], xtab[f].at[pl.ds(r0, SLICE_R)],
            sem_ld))
        stage.append(pltpu.async_copy(
            zbuf, acc[f].at[pl.ds(r0, SLICE_R)], sem_ld))
    for d in stage:
        d.wait()
    plsc.subcore_barrier()

    base = wid * E_PER_W

    def issue_loads(k, b):
        off = base + k * CHUNK
        pltpu.async_copy(ei_hbm.at[0, pl.ds(off, CHUNK)], src_v[b], sem_ld)
        pltpu.async_copy(ei_hbm.at[1, pl.ds(off, CHUNK)], dst_v[b], sem_ld)
        pltpu.async_copy(w_hbm.at[pl.ds(off, CHUNK)], w_v[b], sem_ld)

    def wait_loads(b):
        pltpu.make_async_copy(ei_hbm.at[0, pl.ds(0, CHUNK)], src_v[b],
                              sem_ld).wait()
        pltpu.make_async_copy(ei_hbm.at[1, pl.ds(0, CHUNK)], dst_v[b],
                              sem_ld).wait()
        pltpu.make_async_copy(w_hbm.at[pl.ds(0, CHUNK)], w_v[b],
                              sem_ld).wait()

    def issue_gathers(b):
        for f in range(F):
            pltpu.async_copy(xtab[f].at[src_v[b]], col_v[b][f], sem_g)

    def wait_gathers(b):
        for f in range(F):
            pltpu.make_async_copy(xtab[f].at[src_v[b]], col_v[b][f],
                                  sem_g).wait()

    def issue_scatters(b):
        for f in range(F):
            pltpu.async_copy(col_v[b][f], acc[f].at[dst_v[b]], sem_s,
                             add=True)

    def wait_scatters(b):
        for f in range(F):
            pltpu.make_async_copy(col_v[b][f], acc[f].at[dst_v[b]],
                                  sem_s).wait()

    issue_loads(0, 0)
    wait_loads(0)
    issue_gathers(0)

    def pair(k2, carry):
        for b in range(2):
            k = k2 * 2 + b

            @pl.when(k < K_CH - 1)
            def _():
                issue_loads(k + 1, 1 - b)
            wait_gathers(b)

            def scale(i, carry2):
                sl = pl.ds(i * 16, 16)
                wv = w_v[b][sl]
                for f in range(F):
                    col_v[b][f][sl] = col_v[b][f][sl] * wv
                return carry2
            lax.fori_loop(0, CHUNK // 16, scale, 0)
            issue_scatters(b)

            @pl.when(k < K_CH - 1)
            def _():
                wait_loads(1 - b)
                issue_gathers(1 - b)
            wait_scatters(b)
        return carry
    lax.fori_loop(0, K_CH // 2, pair, 0)

    plsc.subcore_barrier()
    for f in range(F):
        pltpu.sync_copy(acc[f].at[pl.ds(r0, SLICE_R)],
                        out_hbm.at[c, f, pl.ds(r0, SLICE_R)])


@functools.partial(
    pl.kernel,
    out_type=jax.ShapeDtypeStruct((NC, N), jnp.float32),
    mesh=_MESH,
    scratch_types=[
        pltpu.VMEM_SHARED((N,), jnp.float32),
        pltpu.VMEM((N,), jnp.float32),
        [pltpu.VMEM((CHUNK,), jnp.int32) for _ in range(2)],
        [pltpu.VMEM((CHUNK,), jnp.int32) for _ in range(2)],
        [pltpu.VMEM((CHUNK,), jnp.float32) for _ in range(2)],
        [pltpu.VMEM((CHUNK,), jnp.float32) for _ in range(2)],
        pltpu.SemaphoreType.DMA,
        pltpu.SemaphoreType.DMA,
    ],
    compiler_params=_PARAMS,
)
def _agg1(ei_hbm, w_hbm, g_hbm, out_hbm,
          acc, gtab, src_v, dst_v, w_v, m_v, sem_ld, sem_s):
    c = lax.axis_index("c")
    s = lax.axis_index("s")
    wid = c * NS + s
    r0 = jnp.minimum(s * SLICE_R, N - SLICE_R)
    gd = pltpu.async_copy(g_hbm.at[pl.ds(0, N)], gtab, sem_ld)
    z16 = jnp.zeros((16,), jnp.float32)

    def zb(i, carry):
        m_v[0][pl.ds(i * 16, 16)] = z16
        return carry
    lax.fori_loop(0, CHUNK // 16, zb, 0)
    nz = SLICE_R // CHUNK + 1

    def zcp(i, carry):
        o2 = jnp.minimum(r0 + i * CHUNK, r0 + SLICE_R - CHUNK)
        pltpu.sync_copy(m_v[0], acc.at[pl.ds(o2, CHUNK)])
        return carry
    lax.fori_loop(0, nz, zcp, 0)
    gd.wait()
    plsc.subcore_barrier()

    base = wid * E_PER_W

    def issue_loads(k, b):
        off = base + k * CHUNK
        pltpu.async_copy(ei_hbm.at[0, pl.ds(off, CHUNK)], src_v[b], sem_ld)
        pltpu.async_copy(ei_hbm.at[1, pl.ds(off, CHUNK)], dst_v[b], sem_ld)
        pltpu.async_copy(w_hbm.at[pl.ds(off, CHUNK)], w_v[b], sem_ld)

    def wait_loads(b):
        pltpu.make_async_copy(ei_hbm.at[0, pl.ds(0, CHUNK)], src_v[b],
                              sem_ld).wait()
        pltpu.make_async_copy(ei_hbm.at[1, pl.ds(0, CHUNK)], dst_v[b],
                              sem_ld).wait()
        pltpu.make_async_copy(w_hbm.at[pl.ds(0, CHUNK)], w_v[b],
                              sem_ld).wait()

    issue_loads(0, 0)
    wait_loads(0)

    def pair(k2, carry):
        for b in range(2):
            k = k2 * 2 + b

            @pl.when(k < K_CH - 1)
            def _():
                issue_loads(k + 1, 1 - b)

            def prod(i, carry2):
                sl = pl.ds(i * 16, 16)
                gv = plsc.load_gather(gtab, [src_v[b][sl]])
                m_v[b][sl] = gv * w_v[b][sl]
                return carry2
            lax.fori_loop(0, CHUNK // 16, prod, 0)
            d = pltpu.async_copy(m_v[b], acc.at[dst_v[b]], sem_s, add=True)

            @pl.when(k < K_CH - 1)
            def _():
                wait_loads(1 - b)
            d.wait()
        return carry
    lax.fori_loop(0, K_CH // 2, pair, 0)

    plsc.subcore_barrier()
    pltpu.sync_copy(acc.at[pl.ds(r0, SLICE_R)],
                    out_hbm.at[c, pl.ds(r0, SLICE_R)])


def _mid_body(p, x, w1, b1, wr1, w2, wr2, b2, g_out, rb_out):
    pv = p[...]
    aggT = pv[0] + pv[1]
    xv = x[...]
    w1v, wr1v = w1[...], wr1[...]
    pre = b1[...][None, :]
    for f in range(F):
        pre = pre + aggT[f, :][:, None] * w1v[f, :][None, :]
        pre = pre + xv[:, f][:, None] * wr1v[f, :][None, :]
    h = jnp.maximum(pre, 0.0)
    g_out[...] = jnp.dot(h, w2[...], preferred_element_type=jnp.float32)[:, 0]
    rb_out[...] = (jnp.dot(h, wr2[...], preferred_element_type=jnp.float32)
                   + b2[...][0])[:, 0]


def _fin_body(sp, rb, out):
    spv = sp[...]
    out[...] = (spv[0] + spv[1] + rb[...])[:, None]


def _full_spec(shape):
    nd = len(shape)
    return pl.BlockSpec(shape, lambda i: (0,) * nd)


_mid = pl.pallas_call(
    _mid_body,
    grid=(GRID,),
    in_specs=[
        pl.BlockSpec((NC, F, BLK), lambda i: (0, 0, i)),
        pl.BlockSpec((BLK, F), lambda i: (i, 0)),
        _full_spec((F, HID)), _full_spec((HID,)), _full_spec((F, HID)),
        _full_spec((HID, 1)), _full_spec((HID, 1)), _full_spec((1,)),
    ],
    out_specs=[pl.BlockSpec((BLK,), lambda i: (i,)),
               pl.BlockSpec((BLK,), lambda i: (i,))],
    out_shape=[jax.ShapeDtypeStruct((NP,), jnp.float32),
               jax.ShapeDtypeStruct((NP,), jnp.float32)],
)

_fin = pl.pallas_call(
    _fin_body,
    grid=(GRID,),
    in_specs=[pl.BlockSpec((NC, BLK), lambda i: (0, i)),
              pl.BlockSpec((BLK,), lambda i: (i,))],
    out_specs=pl.BlockSpec((BLK, 1), lambda i: (i, 0)),
    out_shape=jax.ShapeDtypeStruct((N, 1), jnp.float32),
)


def kernel(x, edge_index, edge_attr, W_rel1, b_rel1, W_root1,
           W_rel2, b_rel2, W_root2):
    ei = edge_index.astype(jnp.int32)
    w = edge_attr.astype(jnp.float32)
    xcols = [x[:, f] for f in range(F)]

    p = _agg4(ei, w, *xcols)
    g, rb = _mid(p, x, W_rel1, b_rel1, W_root1,
                 W_rel2, W_root2, b_rel2)
    sp = _agg1(ei, w, g)
    return _fin(sp, rb)
